# Initial kernel scaffold; baseline (speedup 1.0000x reference)
import functools

import jax
import jax.numpy as jnp
from jax.experimental import pallas as pl
from jax.experimental.pallas import tpu as pltpu

B = 64
C = 6625
T = 80
MAX_LEN = 25
LPAD = 32


def _ace_kernel(starts_ref, lens_ref, x_ref, y_ref, out_ref):
    b = pl.program_id(0)

    xb = x_ref[0]
    m = jnp.max(xb, axis=0, keepdims=True)
    row_ids = jax.lax.broadcasted_iota(jnp.int32, (C, T), 0)
    cand = jnp.where(xb == m, row_ids, C)
    predicts = jnp.min(cand, axis=0, keepdims=True)

    start = starts_ref[b]
    length = lens_ref[b]

    lab = y_ref[pl.ds(start, LPAD), :]
    pos = jax.lax.broadcasted_iota(jnp.int32, (LPAD, 1), 0)
    valid = pos < length
    lab = jnp.where(valid, lab, -1)

    lab_b = jnp.broadcast_to(lab, (LPAD, LPAD))
    eye = (jax.lax.broadcasted_iota(jnp.int32, (LPAD, LPAD), 0)
           == jax.lax.broadcasted_iota(jnp.int32, (LPAD, LPAD), 1))
    lab_row = jnp.sum(jnp.where(eye, lab_b, 0), axis=0, keepdims=True)

    mult = jnp.sum((lab == lab_row).astype(jnp.float32), axis=1, keepdims=True)
    mcnt = jnp.sum((lab == predicts).astype(jnp.float32), axis=1, keepdims=True)

    validf = valid.astype(jnp.float32)
    inv_mult = validf / mult
    sum_nk = jnp.sum(mcnt * inv_mult)

    n_p = jnp.where(sum_nk == 0.0, 1e-5, jnp.maximum(mcnt / sum_nk, 1e-5))
    log_yp = jnp.log(mult) - jnp.log(length.astype(jnp.float32))
    loss_b = jnp.sum(jnp.where(valid, -n_p * log_yp * inv_mult, 0.0))

    @pl.when(b == 0)
    def _():
        out_ref[0, 0] = 0.0

    out_ref[0, 0] += loss_b * (1.0 / B)


@jax.jit
def kernel(x, y, target_lengths):
    ends = jnp.cumsum(target_lengths)
    starts = (ends - target_lengths).astype(jnp.int32)
    y_pad = jnp.zeros((y.shape[0] + LPAD, 1), jnp.int32).at[: y.shape[0], 0].set(y)

    out = pl.pallas_call(
        _ace_kernel,
        grid=(B,),
        in_specs=[
            pl.BlockSpec(memory_space=pltpu.SMEM),
            pl.BlockSpec(memory_space=pltpu.SMEM),
            pl.BlockSpec((1, C, T), lambda b: (b, 0, 0)),
            pl.BlockSpec((y.shape[0] + LPAD, 1), lambda b: (0, 0)),
        ],
        out_specs=pl.BlockSpec((1, 1), lambda b: (0, 0)),
        out_shape=jax.ShapeDtypeStruct((1, 1), jnp.float32),
    )(starts, target_lengths, x, y_pad)
    return out[0, 0]


# speedup vs baseline: 10.9342x; 10.9342x over previous
import functools

import jax
import jax.numpy as jnp
from jax.experimental import pallas as pl
from jax.experimental.pallas import tpu as pltpu

B = 64
C = 6625
T = 80
MAX_LEN = 25
LPAD = 32


def _ace_kernel(starts_ref, lens_ref, x_ref, y_ref, out_ref):
    b = pl.program_id(0)

    xb = x_ref[0]
    m = jnp.max(xb, axis=0, keepdims=True)
    row_ids = jax.lax.broadcasted_iota(jnp.int32, (C, T), 0)
    cand = jnp.where(xb == m, row_ids, C)
    predicts = jnp.min(cand, axis=0, keepdims=True)

    start = starts_ref[b]
    length = lens_ref[b]

    lab = y_ref[pl.ds(start, LPAD), :]
    pos = jax.lax.broadcasted_iota(jnp.int32, (LPAD, 1), 0)
    valid = pos < length
    lab = jnp.where(valid, lab, -1)

    lab_b = jnp.broadcast_to(lab, (LPAD, LPAD))
    eye = (jax.lax.broadcasted_iota(jnp.int32, (LPAD, LPAD), 0)
           == jax.lax.broadcasted_iota(jnp.int32, (LPAD, LPAD), 1))
    lab_row = jnp.sum(jnp.where(eye, lab_b, 0), axis=0, keepdims=True)

    mult = jnp.sum((lab == lab_row).astype(jnp.float32), axis=1, keepdims=True)
    mcnt = jnp.sum((lab == predicts).astype(jnp.float32), axis=1, keepdims=True)

    validf = valid.astype(jnp.float32)
    inv_mult = validf / mult
    sum_nk = jnp.sum(mcnt * inv_mult, keepdims=True)[:, :1]

    n_p = jnp.where(sum_nk == 0.0, 1e-5, jnp.maximum(mcnt / sum_nk, 1e-5))
    log_yp = jnp.log(mult) - jnp.log(length.astype(jnp.float32))
    contrib = jnp.where(valid, -n_p * log_yp * inv_mult, 0.0)
    loss_b = jnp.sum(contrib, keepdims=True)[:, :1]

    @pl.when(b == 0)
    def _():
        out_ref[...] = jnp.zeros((1, 1), jnp.float32)

    out_ref[...] += loss_b * (1.0 / B)


@jax.jit
def kernel(x, y, target_lengths):
    ends = jnp.cumsum(target_lengths)
    starts = (ends - target_lengths).astype(jnp.int32)
    y_pad = jnp.zeros((y.shape[0] + LPAD, 1), jnp.int32).at[: y.shape[0], 0].set(y)

    out = pl.pallas_call(
        _ace_kernel,
        grid=(B,),
        in_specs=[
            pl.BlockSpec(memory_space=pltpu.SMEM),
            pl.BlockSpec(memory_space=pltpu.SMEM),
            pl.BlockSpec((1, C, T), lambda b: (b, 0, 0)),
            pl.BlockSpec((y.shape[0] + LPAD, 1), lambda b: (0, 0)),
        ],
        out_specs=pl.BlockSpec((1, 1), lambda b: (0, 0)),
        out_shape=jax.ShapeDtypeStruct((1, 1), jnp.float32),
    )(starts, target_lengths, x, y_pad)
    return out[0, 0]


# speedup vs baseline: 10.9705x; 1.0033x over previous
import functools

import jax
import jax.numpy as jnp
from jax.experimental import pallas as pl
from jax.experimental.pallas import tpu as pltpu

B = 64
C = 6625
T = 80
MAX_LEN = 25
LPAD = 32


NSTRIPE = 4
CB = 1664


def _ace_kernel(starts_ref, lens_ref, x0_ref, x1_ref, x2_ref, x3_ref, y_ref,
                out_ref):
    b = pl.program_id(0)

    neg_inf = jnp.float32(-jnp.inf)
    xbs = []
    for s, ref in enumerate((x0_ref, x1_ref, x2_ref, x3_ref)):
        xb = ref[0]
        if (s + 1) * CB > C:
            rid = jax.lax.broadcasted_iota(jnp.int32, (CB, T), 0) + s * CB
            xb = jnp.where(rid < C, xb, neg_inf)
        xbs.append(xb)

    m = xbs[0].max(axis=0, keepdims=True)
    for xb in xbs[1:]:
        m = jnp.maximum(m, xb.max(axis=0, keepdims=True))

    predicts = jnp.full((1, T), C, jnp.int32)
    for s, xb in enumerate(xbs):
        rid = jax.lax.broadcasted_iota(jnp.int32, (CB, T), 0) + s * CB
        cand = jnp.where(xb == m, rid, C)
        predicts = jnp.minimum(predicts, cand.min(axis=0, keepdims=True))

    start = starts_ref[b]
    length = lens_ref[b]

    lab = y_ref[pl.ds(start, LPAD), :]
    pos = jax.lax.broadcasted_iota(jnp.int32, (LPAD, 1), 0)
    valid = pos < length
    lab = jnp.where(valid, lab, -1)

    lab_b = jnp.broadcast_to(lab, (LPAD, LPAD))
    eye = (jax.lax.broadcasted_iota(jnp.int32, (LPAD, LPAD), 0)
           == jax.lax.broadcasted_iota(jnp.int32, (LPAD, LPAD), 1))
    lab_row = jnp.sum(jnp.where(eye, lab_b, 0), axis=0, keepdims=True)

    mult = jnp.sum((lab == lab_row).astype(jnp.float32), axis=1, keepdims=True)
    mcnt = jnp.sum((lab == predicts).astype(jnp.float32), axis=1, keepdims=True)

    validf = valid.astype(jnp.float32)
    inv_mult = validf / mult
    sum_nk = jnp.sum(mcnt * inv_mult, keepdims=True)[:, :1]

    n_p = jnp.where(sum_nk == 0.0, 1e-5, jnp.maximum(mcnt / sum_nk, 1e-5))
    log_yp = jnp.log(mult) - jnp.log(length.astype(jnp.float32))
    contrib = jnp.where(valid, -n_p * log_yp * inv_mult, 0.0)
    loss_b = jnp.sum(contrib, keepdims=True)[:, :1]

    @pl.when(b == 0)
    def _():
        out_ref[...] = jnp.zeros((1, 1), jnp.float32)

    out_ref[...] += loss_b * (1.0 / B)


@jax.jit
def kernel(x, y, target_lengths):
    ends = jnp.cumsum(target_lengths)
    starts = (ends - target_lengths).astype(jnp.int32)
    y_pad = jnp.zeros((y.shape[0] + LPAD, 1), jnp.int32).at[: y.shape[0], 0].set(y)

    out = pl.pallas_call(
        _ace_kernel,
        grid=(B,),
        in_specs=[
            pl.BlockSpec(memory_space=pltpu.SMEM),
            pl.BlockSpec(memory_space=pltpu.SMEM),
        ] + [
            pl.BlockSpec((1, CB, T), functools.partial(
                lambda b, s=0: (b, s, 0), s=s))
            for s in range(NSTRIPE)
        ] + [
            pl.BlockSpec((y.shape[0] + LPAD, 1), lambda b: (0, 0)),
        ],
        out_specs=pl.BlockSpec((1, 1), lambda b: (0, 0)),
        out_shape=jax.ShapeDtypeStruct((1, 1), jnp.float32),
    )(starts, target_lengths, x, x, x, x, y_pad)
    return out[0, 0]
